# baseline (speedup 1.0000x reference)
import jax
import jax.numpy as jnp
from jax.experimental import pallas as pl

BATCH_HEADS = 32
SEQ = 4096
D_K = 64
ROUNDS = 4
NB2 = 32
LANES = 128

S_BLK = 2048


def _lsh_kernel(x_ref, w_ref, o_ref):
    x = x_ref[0]
    w = w_ref[0]
    h = jnp.dot(x, w, preferred_element_type=jnp.float32)
    for r in range(ROUNDS):
        hr = h[:, r * LANES:(r + 1) * LANES]
        o_ref[0, r, :] = jnp.argmax(hr, axis=-1).astype(jnp.int32)


@jax.jit
def kernel(inp, rand_matrix):
    w = rand_matrix
    pad = jnp.zeros((BATCH_HEADS, D_K, ROUNDS, LANES - 2 * NB2), jnp.float32)
    wcat = jnp.concatenate([w, -w, pad], axis=-1)
    wcat = wcat.reshape(BATCH_HEADS, D_K, ROUNDS * LANES)
    out = pl.pallas_call(
        _lsh_kernel,
        grid=(BATCH_HEADS, SEQ // S_BLK),
        in_specs=[
            pl.BlockSpec((1, S_BLK, D_K), lambda b, s: (b, s, 0)),
            pl.BlockSpec((1, D_K, ROUNDS * LANES), lambda b, s: (b, 0, 0)),
        ],
        out_specs=pl.BlockSpec((1, ROUNDS, S_BLK), lambda b, s: (b, 0, s)),
        out_shape=jax.ShapeDtypeStruct((BATCH_HEADS, ROUNDS, SEQ), jnp.int32),
    )(inp, wcat)
    return out.transpose(0, 2, 1)


# speedup vs baseline: 1.5747x; 1.5747x over previous
import jax
import jax.numpy as jnp
from jax.experimental import pallas as pl
from jax.experimental.pallas import tpu as pltpu

BATCH_HEADS = 32
SEQ = 4096
D_K = 64
ROUNDS = 4
NB2 = 32
LANES = 128

S_BLK = 4096


def _lsh_kernel(x_ref, w_ref, o_ref):
    x = x_ref[0]
    w = w_ref[0]
    h = jnp.dot(x, w, preferred_element_type=jnp.float32)
    cols = []
    for r in range(ROUNDS):
        hr = h[:, r * LANES:(r + 1) * LANES]
        cols.append(jnp.argmax(hr, axis=-1).astype(jnp.int32))
    o_ref[0] = jnp.stack(cols, axis=-1)


@jax.jit
def kernel(inp, rand_matrix):
    w = rand_matrix
    pad = jnp.zeros((BATCH_HEADS, D_K, ROUNDS, LANES - 2 * NB2), jnp.float32)
    wcat = jnp.concatenate([w, -w, pad], axis=-1)
    wcat = wcat.reshape(BATCH_HEADS, D_K, ROUNDS * LANES)
    out = pl.pallas_call(
        _lsh_kernel,
        grid=(BATCH_HEADS, SEQ // S_BLK),
        in_specs=[
            pl.BlockSpec((1, S_BLK, D_K), lambda b, s: (b, s, 0)),
            pl.BlockSpec((1, D_K, ROUNDS * LANES), lambda b, s: (b, 0, 0)),
        ],
        out_specs=pl.BlockSpec((1, S_BLK, ROUNDS), lambda b, s: (b, s, 0)),
        out_shape=jax.ShapeDtypeStruct((BATCH_HEADS, SEQ, ROUNDS), jnp.int32),
        compiler_params=pltpu.CompilerParams(
            dimension_semantics=("parallel", "parallel"),
        ),
    )(inp, wcat)
    return out
